# baseline (speedup 1.0000x reference)
import functools

import jax
import jax.numpy as jnp
from jax import lax
from jax.experimental import pallas as pl
from jax.experimental.pallas import tpu as pltpu
from jax.experimental.pallas import tpu_sc as plsc

NSC = 2
NTILE = 16
LANES = 16


def _feat_body(P, *refs):
    h_refs = refs[:P]
    W_ref, al_ref, ar_ref = refs[P:P + 3]
    feat_ref, el_ref, er_ref = refs[P + 3:]
    half = pl.program_id(2)
    h = jnp.concatenate([r[0] for r in h_refs], axis=1)
    feat = jnp.dot(h, W_ref[0, 0], preferred_element_type=jnp.float32)
    feat_ref[0] = feat
    elp = feat @ al_ref[0, 0, 0, 0]
    erp = feat @ ar_ref[0, 0, 0, 0]

    @pl.when(half == 0)
    def _():
        el_ref[0, 0] = elp
        er_ref[0, 0] = erp

    @pl.when(half != 0)
    def _():
        el_ref[0, 0] = el_ref[0, 0] + elp
        er_ref[0, 0] = er_ref[0, 0] + erp


def _feat_el_er(h3, W, al, ar, n_pad, nhalf):
    P = h3.shape[0]
    d0 = h3.shape[2]
    d = P * d0
    heads, f = al.shape
    fa = f // nhalf
    Ws = W.reshape(d, heads, nhalf, fa).transpose(1, 2, 0, 3)
    als = al.reshape(heads, nhalf, 1, 1, fa)
    ars = ar.reshape(heads, nhalf, 1, 1, fa)
    blk = 2048
    grid = (n_pad // blk, heads, nhalf)
    h_specs = [
        pl.BlockSpec((1, blk, d0), functools.partial(
            lambda p, i, hh, q: (p, i, 0), p))
        for p in range(P)
    ]
    feat, el, er = pl.pallas_call(
        functools.partial(_feat_body, P),
        grid=grid,
        in_specs=h_specs + [
            pl.BlockSpec((1, 1, d, fa), lambda i, hh, q: (hh, q, 0, 0)),
            pl.BlockSpec((1, 1, 1, 1, fa), lambda i, hh, q: (hh, q, 0, 0, 0)),
            pl.BlockSpec((1, 1, 1, 1, fa), lambda i, hh, q: (hh, q, 0, 0, 0)),
        ],
        out_specs=(
            pl.BlockSpec((1, blk, fa),
                         lambda i, hh, q: (hh * nhalf + q, i, 0)),
            pl.BlockSpec((1, 1, blk), lambda i, hh, q: (hh, 0, i)),
            pl.BlockSpec((1, 1, blk), lambda i, hh, q: (hh, 0, i)),
        ),
        out_shape=(
            jax.ShapeDtypeStruct((heads * nhalf, n_pad, fa), jnp.float32),
            jax.ShapeDtypeStruct((heads, 1, n_pad), jnp.float32),
            jax.ShapeDtypeStruct((heads, 1, n_pad), jnp.float32),
        ),
    )(*([h3] * P), Ws, als, ars)
    return (feat.reshape(heads * nhalf * n_pad, fa),
            el.reshape(heads, n_pad), er.reshape(heads, n_pad))


def _sc_body(n_pad, e_real, nchunk, cs, fa, nhalf,
             feat_hbm, el_hbm, er_hbm, b_hbm, src_hbm, dst_hbm, out_hbm,
             src_v, dst_v, el_v, er_v, ee_v, rows_v, rows2_v, fin_v, idx_v,
             idx2_v, bias_v, den_sl, acc_s, den_s, sem, sem2, sem3, sem4):
    c = lax.axis_index("c")
    s = lax.axis_index("s")
    ept = nchunk * cs
    rows_per_tile = n_pad // NTILE
    row0 = s * rows_per_tile
    fpl = fa // LANES
    cpl = cs // LANES

    pltpu.sync_copy(el_hbm.at[c], el_v)
    pltpu.sync_copy(er_hbm.at[c], er_v)
    pltpu.sync_copy(src_hbm.at[s], src_v)
    pltpu.sync_copy(dst_hbm.at[s], dst_v)

    zero16 = jnp.zeros((LANES,), jnp.float32)

    def _zden(k, _):
        den_sl[pl.ds(k * LANES, LANES)] = zero16
        return 0
    lax.fori_loop(0, rows_per_tile // LANES, _zden, 0)
    pltpu.sync_copy(den_sl, den_s.at[pl.ds(row0, rows_per_tile)])

    def _mx(i, carry):
        mel, mer = carry
        mel = jnp.maximum(mel, el_v[pl.ds(i * LANES, LANES)])
        mer = jnp.maximum(mer, er_v[pl.ds(i * LANES, LANES)])
        return mel, mer
    neg = jnp.full((LANES,), -3.0e38, jnp.float32)
    mel, mer = lax.fori_loop(0, n_pad // LANES, _mx, (neg, neg))
    lane0 = lax.iota(jnp.int32, LANES)
    for sh in (8, 4, 2, 1):
        perm = lane0 ^ sh
        mel = jnp.maximum(mel, mel[perm])
        mer = jnp.maximum(mer, mer[perm])
    msc = mel[0] + mer[0]
    msc = jnp.where(msc > 0, msc, 0.2 * msc)

    plsc.subcore_barrier()

    lane = lax.iota(jnp.int32, LANES)

    def _edges(j, _):
        for k in range(cpl):
            s16 = src_v[j, pl.ds(k * LANES, LANES)]
            d16 = dst_v[j, pl.ds(k * LANES, LANES)]
            e16 = (plsc.load_gather(el_v, [s16])
                   + plsc.load_gather(er_v, [d16]))
            e16 = jnp.where(e16 > 0, e16, 0.2 * e16)
            ee16 = jnp.exp(e16 - msc)
            gid = s * ept + j * cs + k * LANES + lane
            ee16 = jnp.where(gid < e_real, ee16, 0.0)
            ee_v[pl.ds(j * cs + k * LANES, LANES)] = ee16
        pltpu.async_copy(ee_v.at[pl.ds(j * cs, cs)],
                         den_s.at[dst_v.at[j]], sem2, add=True)
        return 0
    lax.fori_loop(0, nchunk, _edges, 0)

    def _edges_drain(j, _):
        pltpu.make_async_copy(ee_v.at[pl.ds(j * cs, cs)],
                              den_s.at[dst_v.at[j]], sem2).wait()
        return 0
    lax.fori_loop(0, nchunk, _edges_drain, 0)

    plsc.subcore_barrier()
    pltpu.sync_copy(den_s.at[pl.ds(row0, rows_per_tile)], den_sl)

    def _half(half, _):
        plane = c * nhalf + half

        def _zrow(r, _):
            for g in range(fpl):
                fin_v[r, pl.ds(g * LANES, LANES)] = zero16
            return 0
        lax.fori_loop(0, cs, _zrow, 0)
        for i in range(rows_per_tile // cs):
            pltpu.sync_copy(fin_v, acc_s.at[pl.ds(row0 + i * cs, cs)])
        plsc.subcore_barrier()

        def _bidx(j, buf):
            for k in range(cpl):
                buf[pl.ds(k * LANES, LANES)] = (
                    src_v[j, pl.ds(k * LANES, LANES)] + plane * n_pad)

        def _scale_rows(j, rows):
            @plsc.parallel_loop(0, cpl)
            def _scale(kk):
                ee16 = ee_v[pl.ds(j * cs + kk * LANES, LANES)]
                for r16 in range(LANES):
                    w = ee16[r16]
                    r = kk * LANES + r16
                    for g in range(fpl):
                        rows[r, pl.ds(g * LANES, LANES)] = (
                            rows[r, pl.ds(g * LANES, LANES)] * w)

        def _wait_scat(rows, ssem):
            pltpu.make_async_copy(rows, acc_s.at[dst_v.at[0]], ssem).wait()

        _bidx(0, idx_v)
        pltpu.async_copy(feat_hbm.at[idx_v], rows_v, sem)

        def _agg2(jj, _):
            j0 = 2 * jj
            j1 = j0 + 1

            @pl.when(jj > 0)
            def _():
                _wait_scat(rows2_v, sem4)
            _bidx(j1, idx2_v)
            pltpu.async_copy(feat_hbm.at[idx2_v], rows2_v, sem2)
            pltpu.make_async_copy(feat_hbm.at[idx_v], rows_v, sem).wait()
            _scale_rows(j0, rows_v)
            pltpu.async_copy(rows_v, acc_s.at[dst_v.at[j0]], sem3, add=True)

            @pl.when(j0 + 2 < nchunk)
            def _():
                _wait_scat(rows_v, sem3)
                _bidx(j0 + 2, idx_v)
                pltpu.async_copy(feat_hbm.at[idx_v], rows_v, sem)
            pltpu.make_async_copy(feat_hbm.at[idx2_v], rows2_v, sem2).wait()
            _scale_rows(j1, rows2_v)
            pltpu.async_copy(rows2_v, acc_s.at[dst_v.at[j1]], sem4, add=True)
            return 0
        lax.fori_loop(0, nchunk // 2, _agg2, 0)
        if nchunk % 2 == 1:
            pltpu.make_async_copy(feat_hbm.at[idx_v], rows_v, sem).wait()
            _scale_rows(nchunk - 1, rows_v)
            pltpu.async_copy(rows_v, acc_s.at[dst_v.at[nchunk - 1]],
                             sem3, add=True)
            _wait_scat(rows_v, sem3)
        else:
            _wait_scat(rows_v, sem3)
        _wait_scat(rows2_v, sem4)

        plsc.subcore_barrier()

        pltpu.sync_copy(b_hbm.at[plane], bias_v)
        for i in range(rows_per_tile // cs):
            pltpu.sync_copy(acc_s.at[pl.ds(row0 + i * cs, cs)], fin_v)

            @plsc.parallel_loop(0, cpl)
            def _fin(kk):
                den16 = den_sl[pl.ds(i * cs + kk * LANES, LANES)]
                inv16 = 1.0 / jnp.where(den16 != 0.0, den16, 1.0)
                for r16 in range(LANES):
                    inv = inv16[r16]
                    r = kk * LANES + r16
                    for g in range(fpl):
                        v = (fin_v[r, pl.ds(g * LANES, LANES)] * inv
                             + bias_v[pl.ds(g * LANES, LANES)])
                        fin_v[r, pl.ds(g * LANES, LANES)] = jnp.where(
                            v > 0, v, jnp.exp(jnp.minimum(v, 0.0)) - 1.0)
            pltpu.sync_copy(fin_v,
                            out_hbm.at[plane, pl.ds(row0 + i * cs, cs)])
        plsc.subcore_barrier()
        return 0
    lax.fori_loop(0, nhalf, _half, 0)


def _sc_layer(feat2d, el, er, b2, src3, dst3, n_pad, e_real, nchunk, cs,
              nhalf):
    fa = feat2d.shape[1]
    mesh = plsc.VectorSubcoreMesh(core_axis_name="c", subcore_axis_name="s")
    body = functools.partial(_sc_body, n_pad, e_real, nchunk, cs, fa, nhalf)
    return pl.kernel(
        body,
        out_type=jax.ShapeDtypeStruct((NSC * nhalf, n_pad, fa), jnp.float32),
        mesh=mesh,
        compiler_params=pltpu.CompilerParams(
            needs_layout_passes=False, use_tc_tiling_on_sc=False),
        scratch_types=[
            pltpu.VMEM((nchunk, cs), jnp.int32),
            pltpu.VMEM((nchunk, cs), jnp.int32),
            pltpu.VMEM((n_pad,), jnp.float32),
            pltpu.VMEM((n_pad,), jnp.float32),
            pltpu.VMEM((nchunk * cs,), jnp.float32),
            pltpu.VMEM((cs, fa), jnp.float32),
            pltpu.VMEM((cs, fa), jnp.float32),
            pltpu.VMEM((cs, fa), jnp.float32),
            pltpu.VMEM((cs,), jnp.int32),
            pltpu.VMEM((cs,), jnp.int32),
            pltpu.VMEM((fa,), jnp.float32),
            pltpu.VMEM((n_pad // NTILE,), jnp.float32),
            pltpu.VMEM_SHARED((n_pad, fa), jnp.float32),
            pltpu.VMEM_SHARED((n_pad,), jnp.float32),
            pltpu.SemaphoreType.DMA,
            pltpu.SemaphoreType.DMA,
            pltpu.SemaphoreType.DMA,
            pltpu.SemaphoreType.DMA,
        ],
    )(feat2d, el, er, b2, src3, dst3)


def _pool_body(n_real, blk, nblk, P, *refs):
    h_refs = refs[:P]
    (gw1_ref, gb1_ref, gw2_ref, gb2_ref,
     mu_w_ref, mu_b_ref, lv_w_ref, lv_b_ref) = refs[P:P + 8]
    mu_ref, lv_ref = refs[P + 8:P + 10]
    vacc, sacc, macc = refs[P + 10:]
    i = pl.program_id(0)
    h = jnp.concatenate([r[0] for r in h_refs], axis=1)
    g1 = jnp.maximum(
        jnp.dot(h, gw1_ref[...], preferred_element_type=jnp.float32)
        + gb1_ref[...], 0.0)
    gate = jnp.dot(g1, gw2_ref[...], preferred_element_type=jnp.float32) \
        + gb2_ref[...]
    rows = lax.broadcasted_iota(jnp.int32, gate.shape, 0) + i * blk
    gate = jnp.where(rows < n_real, gate, -3.0e38)
    mb = jnp.max(gate)

    @pl.when(i == 0)
    def _():
        macc[0] = mb
        sacc[0] = 0.0
        vacc[...] = jnp.zeros_like(vacc)

    m_old = macc[0]
    m_new = jnp.maximum(m_old, mb)
    macc[0] = m_new
    scale = jnp.exp(m_old - m_new)
    w = jnp.where(rows < n_real, jnp.exp(gate - m_new), 0.0)
    sacc[0] = sacc[0] * scale + jnp.sum(w)
    vacc[...] = vacc[...] * scale + jnp.sum(w * h, axis=0, keepdims=True)

    @pl.when(i == nblk - 1)
    def _():
        emb = vacc[...] / sacc[0]
        mu_ref[...] = (
            jnp.dot(emb, mu_w_ref[...], preferred_element_type=jnp.float32)
            + mu_b_ref[...])
        lv_ref[...] = (
            jnp.dot(emb, lv_w_ref[...], preferred_element_type=jnp.float32)
            + lv_b_ref[...])


def _pool_heads(h, gw1, gb1, gw2, gb2, mu_w, mu_b, lv_w, lv_b, n_real):
    P, n_pad, dp = h.shape
    blk = 2048
    nblk = n_pad // blk
    wspec = [
        pl.BlockSpec(a.shape, lambda i: (0,) * a.ndim)
        for a in (gw1, gb1[None, :], gw2, gb2[None, :])
    ]
    hspecs = [
        pl.BlockSpec((1, blk, dp), functools.partial(
            lambda p, i: (p, i, 0), p))
        for p in range(P)
    ]
    return pl.pallas_call(
        functools.partial(_pool_body, n_real, blk, nblk, P),
        grid=(nblk,),
        in_specs=hspecs + wspec + [
            pl.BlockSpec(a.shape, lambda i: (0,) * a.ndim)
            for a in (mu_w, mu_b[None, :], lv_w, lv_b[None, :])
        ],
        out_specs=(
            pl.BlockSpec((1, 64), lambda i: (0, 0)),
            pl.BlockSpec((1, 64), lambda i: (0, 0)),
        ),
        out_shape=(
            jax.ShapeDtypeStruct((1, 64), jnp.float32),
            jax.ShapeDtypeStruct((1, 64), jnp.float32),
        ),
        scratch_shapes=[
            pltpu.VMEM((1, P * dp), jnp.float32),
            pltpu.SMEM((1,), jnp.float32),
            pltpu.SMEM((1,), jnp.float32),
        ],
    )(*([h] * P), gw1, gb1[None, :], gw2, gb2[None, :],
      mu_w, mu_b[None, :], lv_w, lv_b[None, :])


def kernel(x, W1, al1, ar1, b1, W2, al2, ar2, b2, W3, al3, ar3, b3,
           gw1, gb1, gw2, gb2, mu_w, mu_b, lv_w, lv_b, edge_index):
    n = x.shape[0]
    e = edge_index.shape[1]
    cs = 128
    blk = NTILE * cs
    n_pad = ((n + blk - 1) // blk) * blk
    ept = ((e + NTILE * cs - 1) // (NTILE * cs)) * cs
    nchunk = ept // cs
    e_pad = ept * NTILE

    src = jnp.concatenate(
        [edge_index[0], jnp.zeros((e_pad - e,), edge_index.dtype)]
    ).astype(jnp.int32).reshape(NTILE, nchunk, cs)
    dst = jnp.concatenate(
        [edge_index[1], jnp.zeros((e_pad - e,), edge_index.dtype)]
    ).astype(jnp.int32).reshape(NTILE, nchunk, cs)

    xp = jnp.pad(x, ((0, n_pad - n), (0, 0)))
    d0 = x.shape[1] // 2
    h = jnp.stack([xp[:, :d0], xp[:, d0:]])
    for W, al, ar, b, nhalf in ((W1, al1, ar1, b1, 1), (W2, al2, ar2, b2, 2),
                                (W3, al3, ar3, b3, 4)):
        f = al.shape[1]
        fa = f // nhalf
        feat2d, el, er = _feat_el_er(h, W, al, ar, n_pad, nhalf)
        h = _sc_layer(feat2d, el, er, b.reshape(NSC * nhalf, fa), src, dst,
                      n_pad, e, nchunk, cs, nhalf)
    return _pool_heads(h, gw1, gb1, gw2, gb2, mu_w, mu_b, lv_w, lv_b, n)


# speedup vs baseline: 1.0054x; 1.0054x over previous
import functools

import jax
import jax.numpy as jnp
from jax import lax
from jax.experimental import pallas as pl
from jax.experimental.pallas import tpu as pltpu
from jax.experimental.pallas import tpu_sc as plsc

NSC = 2
NTILE = 16
LANES = 16


def _feat_body(P, *refs):
    h_refs = refs[:P]
    W_ref, al_ref, ar_ref = refs[P:P + 3]
    feat_ref, el_ref, er_ref = refs[P + 3:]
    half = pl.program_id(2)
    h = jnp.concatenate([r[0] for r in h_refs], axis=1)
    feat = jnp.dot(h, W_ref[0, 0], preferred_element_type=jnp.float32)
    feat_ref[0] = feat
    elp = feat @ al_ref[0, 0, 0, 0]
    erp = feat @ ar_ref[0, 0, 0, 0]

    @pl.when(half == 0)
    def _():
        el_ref[0, 0] = elp
        er_ref[0, 0] = erp

    @pl.when(half != 0)
    def _():
        el_ref[0, 0] = el_ref[0, 0] + elp
        er_ref[0, 0] = er_ref[0, 0] + erp


def _feat_el_er(h3, W, al, ar, n_pad, nhalf):
    P = h3.shape[0]
    d0 = h3.shape[2]
    d = P * d0
    heads, f = al.shape
    fa = f // nhalf
    Ws = W.reshape(d, heads, nhalf, fa).transpose(1, 2, 0, 3)
    als = al.reshape(heads, nhalf, 1, 1, fa)
    ars = ar.reshape(heads, nhalf, 1, 1, fa)
    blk = 2048
    grid = (n_pad // blk, heads, nhalf)
    h_specs = [
        pl.BlockSpec((1, blk, d0), functools.partial(
            lambda p, i, hh, q: (p, i, 0), p))
        for p in range(P)
    ]
    feat, el, er = pl.pallas_call(
        functools.partial(_feat_body, P),
        grid=grid,
        in_specs=h_specs + [
            pl.BlockSpec((1, 1, d, fa), lambda i, hh, q: (hh, q, 0, 0)),
            pl.BlockSpec((1, 1, 1, 1, fa), lambda i, hh, q: (hh, q, 0, 0, 0)),
            pl.BlockSpec((1, 1, 1, 1, fa), lambda i, hh, q: (hh, q, 0, 0, 0)),
        ],
        out_specs=(
            pl.BlockSpec((1, blk, fa),
                         lambda i, hh, q: (hh * nhalf + q, i, 0)),
            pl.BlockSpec((1, 1, blk), lambda i, hh, q: (hh, 0, i)),
            pl.BlockSpec((1, 1, blk), lambda i, hh, q: (hh, 0, i)),
        ),
        out_shape=(
            jax.ShapeDtypeStruct((heads * nhalf, n_pad, fa), jnp.float32),
            jax.ShapeDtypeStruct((heads, 1, n_pad), jnp.float32),
            jax.ShapeDtypeStruct((heads, 1, n_pad), jnp.float32),
        ),
    )(*([h3] * P), Ws, als, ars)
    return (feat.reshape(heads * nhalf * n_pad, fa),
            el.reshape(heads, n_pad), er.reshape(heads, n_pad))


def _sc_body(n_pad, e_real, nchunk, cs, fa, nhalf,
             feat_hbm, el_hbm, er_hbm, b_hbm, src_hbm, dst_hbm, out_hbm,
             src_v, dst_v, el_v, er_v, ee_v, rows_v, rows2_v, fin_v, idx_v,
             idx2_v, bias_v, den_sl, acc_s, den_s, sem, sem2):
    c = lax.axis_index("c")
    s = lax.axis_index("s")
    ept = nchunk * cs
    rows_per_tile = n_pad // NTILE
    row0 = s * rows_per_tile
    fpl = fa // LANES
    cpl = cs // LANES

    pltpu.sync_copy(el_hbm.at[c], el_v)
    pltpu.sync_copy(er_hbm.at[c], er_v)
    pltpu.sync_copy(src_hbm.at[s], src_v)
    pltpu.sync_copy(dst_hbm.at[s], dst_v)

    zero16 = jnp.zeros((LANES,), jnp.float32)

    def _zden(k, _):
        den_sl[pl.ds(k * LANES, LANES)] = zero16
        return 0
    lax.fori_loop(0, rows_per_tile // LANES, _zden, 0)
    pltpu.sync_copy(den_sl, den_s.at[pl.ds(row0, rows_per_tile)])

    def _mx(i, carry):
        mel, mer = carry
        mel = jnp.maximum(mel, el_v[pl.ds(i * LANES, LANES)])
        mer = jnp.maximum(mer, er_v[pl.ds(i * LANES, LANES)])
        return mel, mer
    neg = jnp.full((LANES,), -3.0e38, jnp.float32)
    mel, mer = lax.fori_loop(0, n_pad // LANES, _mx, (neg, neg))
    lane0 = lax.iota(jnp.int32, LANES)
    for sh in (8, 4, 2, 1):
        perm = lane0 ^ sh
        mel = jnp.maximum(mel, mel[perm])
        mer = jnp.maximum(mer, mer[perm])
    msc = mel[0] + mer[0]
    msc = jnp.where(msc > 0, msc, 0.2 * msc)

    plsc.subcore_barrier()

    lane = lax.iota(jnp.int32, LANES)

    def _edges(j, _):
        for k in range(cpl):
            s16 = src_v[j, pl.ds(k * LANES, LANES)]
            d16 = dst_v[j, pl.ds(k * LANES, LANES)]
            e16 = (plsc.load_gather(el_v, [s16])
                   + plsc.load_gather(er_v, [d16]))
            e16 = jnp.where(e16 > 0, e16, 0.2 * e16)
            ee16 = jnp.exp(e16 - msc)
            gid = s * ept + j * cs + k * LANES + lane
            ee16 = jnp.where(gid < e_real, ee16, 0.0)
            ee_v[pl.ds(j * cs + k * LANES, LANES)] = ee16
        pltpu.async_copy(ee_v.at[pl.ds(j * cs, cs)],
                         den_s.at[dst_v.at[j]], sem2, add=True)
        return 0
    lax.fori_loop(0, nchunk, _edges, 0)

    def _edges_drain(j, _):
        pltpu.make_async_copy(ee_v.at[pl.ds(j * cs, cs)],
                              den_s.at[dst_v.at[j]], sem2).wait()
        return 0
    lax.fori_loop(0, nchunk, _edges_drain, 0)

    plsc.subcore_barrier()
    pltpu.sync_copy(den_s.at[pl.ds(row0, rows_per_tile)], den_sl)

    def _half(half, _):
        plane = c * nhalf + half

        def _zrow(r, _):
            for g in range(fpl):
                fin_v[r, pl.ds(g * LANES, LANES)] = zero16
            return 0
        lax.fori_loop(0, cs, _zrow, 0)
        for i in range(rows_per_tile // cs):
            pltpu.sync_copy(fin_v, acc_s.at[pl.ds(row0 + i * cs, cs)])
        plsc.subcore_barrier()

        def _bidx(j, buf):
            for k in range(cpl):
                buf[pl.ds(k * LANES, LANES)] = (
                    src_v[j, pl.ds(k * LANES, LANES)] + plane * n_pad)

        def _do_chunk(j, rows, idx, dsem):
            pltpu.make_async_copy(feat_hbm.at[idx], rows, dsem).wait()

            @plsc.parallel_loop(0, cpl)
            def _scale(kk):
                ee16 = ee_v[pl.ds(j * cs + kk * LANES, LANES)]
                for r16 in range(LANES):
                    w = ee16[r16]
                    r = kk * LANES + r16
                    for g in range(fpl):
                        rows[r, pl.ds(g * LANES, LANES)] = (
                            rows[r, pl.ds(g * LANES, LANES)] * w)
            pltpu.sync_copy(rows, acc_s.at[dst_v.at[j]], add=True)

        _bidx(0, idx_v)
        pltpu.async_copy(feat_hbm.at[idx_v], rows_v, sem)

        def _agg2(jj, _):
            j0 = 2 * jj
            j1 = j0 + 1
            _bidx(j1, idx2_v)
            pltpu.async_copy(feat_hbm.at[idx2_v], rows2_v, sem2)
            _do_chunk(j0, rows_v, idx_v, sem)

            @pl.when(j0 + 2 < nchunk)
            def _():
                _bidx(j0 + 2, idx_v)
                pltpu.async_copy(feat_hbm.at[idx_v], rows_v, sem)
            _do_chunk(j1, rows2_v, idx2_v, sem2)
            return 0
        lax.fori_loop(0, nchunk // 2, _agg2, 0)
        if nchunk % 2 == 1:
            _do_chunk(nchunk - 1, rows_v, idx_v, sem)

        plsc.subcore_barrier()

        pltpu.sync_copy(b_hbm.at[plane], bias_v)
        for i in range(rows_per_tile // cs):
            pltpu.sync_copy(acc_s.at[pl.ds(row0 + i * cs, cs)], fin_v)

            @plsc.parallel_loop(0, cpl)
            def _fin(kk):
                den16 = den_sl[pl.ds(i * cs + kk * LANES, LANES)]
                inv16 = 1.0 / jnp.where(den16 != 0.0, den16, 1.0)
                for r16 in range(LANES):
                    inv = inv16[r16]
                    r = kk * LANES + r16
                    for g in range(fpl):
                        v = (fin_v[r, pl.ds(g * LANES, LANES)] * inv
                             + bias_v[pl.ds(g * LANES, LANES)])
                        fin_v[r, pl.ds(g * LANES, LANES)] = jnp.where(
                            v > 0, v, jnp.exp(jnp.minimum(v, 0.0)) - 1.0)
            pltpu.sync_copy(fin_v,
                            out_hbm.at[plane, pl.ds(row0 + i * cs, cs)])
        plsc.subcore_barrier()
        return 0
    lax.fori_loop(0, nhalf, _half, 0)


def _sc_layer(feat2d, el, er, b2, src3, dst3, n_pad, e_real, nchunk, cs,
              nhalf):
    fa = feat2d.shape[1]
    mesh = plsc.VectorSubcoreMesh(core_axis_name="c", subcore_axis_name="s")
    body = functools.partial(_sc_body, n_pad, e_real, nchunk, cs, fa, nhalf)
    return pl.kernel(
        body,
        out_type=jax.ShapeDtypeStruct((NSC * nhalf, n_pad, fa), jnp.float32),
        mesh=mesh,
        compiler_params=pltpu.CompilerParams(
            needs_layout_passes=False, use_tc_tiling_on_sc=False),
        scratch_types=[
            pltpu.VMEM((nchunk, cs), jnp.int32),
            pltpu.VMEM((nchunk, cs), jnp.int32),
            pltpu.VMEM((n_pad,), jnp.float32),
            pltpu.VMEM((n_pad,), jnp.float32),
            pltpu.VMEM((nchunk * cs,), jnp.float32),
            pltpu.VMEM((cs, fa), jnp.float32),
            pltpu.VMEM((cs, fa), jnp.float32),
            pltpu.VMEM((cs, fa), jnp.float32),
            pltpu.VMEM((cs,), jnp.int32),
            pltpu.VMEM((cs,), jnp.int32),
            pltpu.VMEM((fa,), jnp.float32),
            pltpu.VMEM((n_pad // NTILE,), jnp.float32),
            pltpu.VMEM_SHARED((n_pad, fa), jnp.float32),
            pltpu.VMEM_SHARED((n_pad,), jnp.float32),
            pltpu.SemaphoreType.DMA,
            pltpu.SemaphoreType.DMA,
        ],
    )(feat2d, el, er, b2, src3, dst3)


def _pool_body(n_real, blk, nblk, P, *refs):
    h_refs = refs[:P]
    (gw1_ref, gb1_ref, gw2_ref, gb2_ref,
     mu_w_ref, mu_b_ref, lv_w_ref, lv_b_ref) = refs[P:P + 8]
    mu_ref, lv_ref = refs[P + 8:P + 10]
    vacc, sacc, macc = refs[P + 10:]
    i = pl.program_id(0)
    h = jnp.concatenate([r[0] for r in h_refs], axis=1)
    g1 = jnp.maximum(
        jnp.dot(h, gw1_ref[...], preferred_element_type=jnp.float32)
        + gb1_ref[...], 0.0)
    gate = jnp.dot(g1, gw2_ref[...], preferred_element_type=jnp.float32) \
        + gb2_ref[...]
    rows = lax.broadcasted_iota(jnp.int32, gate.shape, 0) + i * blk
    gate = jnp.where(rows < n_real, gate, -3.0e38)
    mb = jnp.max(gate)

    @pl.when(i == 0)
    def _():
        macc[0] = mb
        sacc[0] = 0.0
        vacc[...] = jnp.zeros_like(vacc)

    m_old = macc[0]
    m_new = jnp.maximum(m_old, mb)
    macc[0] = m_new
    scale = jnp.exp(m_old - m_new)
    w = jnp.where(rows < n_real, jnp.exp(gate - m_new), 0.0)
    sacc[0] = sacc[0] * scale + jnp.sum(w)
    vacc[...] = vacc[...] * scale + jnp.sum(w * h, axis=0, keepdims=True)

    @pl.when(i == nblk - 1)
    def _():
        emb = vacc[...] / sacc[0]
        mu_ref[...] = (
            jnp.dot(emb, mu_w_ref[...], preferred_element_type=jnp.float32)
            + mu_b_ref[...])
        lv_ref[...] = (
            jnp.dot(emb, lv_w_ref[...], preferred_element_type=jnp.float32)
            + lv_b_ref[...])


def _pool_heads(h, gw1, gb1, gw2, gb2, mu_w, mu_b, lv_w, lv_b, n_real):
    P, n_pad, dp = h.shape
    blk = 2048
    nblk = n_pad // blk
    wspec = [
        pl.BlockSpec(a.shape, lambda i: (0,) * a.ndim)
        for a in (gw1, gb1[None, :], gw2, gb2[None, :])
    ]
    hspecs = [
        pl.BlockSpec((1, blk, dp), functools.partial(
            lambda p, i: (p, i, 0), p))
        for p in range(P)
    ]
    return pl.pallas_call(
        functools.partial(_pool_body, n_real, blk, nblk, P),
        grid=(nblk,),
        in_specs=hspecs + wspec + [
            pl.BlockSpec(a.shape, lambda i: (0,) * a.ndim)
            for a in (mu_w, mu_b[None, :], lv_w, lv_b[None, :])
        ],
        out_specs=(
            pl.BlockSpec((1, 64), lambda i: (0, 0)),
            pl.BlockSpec((1, 64), lambda i: (0, 0)),
        ),
        out_shape=(
            jax.ShapeDtypeStruct((1, 64), jnp.float32),
            jax.ShapeDtypeStruct((1, 64), jnp.float32),
        ),
        scratch_shapes=[
            pltpu.VMEM((1, P * dp), jnp.float32),
            pltpu.SMEM((1,), jnp.float32),
            pltpu.SMEM((1,), jnp.float32),
        ],
    )(*([h] * P), gw1, gb1[None, :], gw2, gb2[None, :],
      mu_w, mu_b[None, :], lv_w, lv_b[None, :])


def kernel(x, W1, al1, ar1, b1, W2, al2, ar2, b2, W3, al3, ar3, b3,
           gw1, gb1, gw2, gb2, mu_w, mu_b, lv_w, lv_b, edge_index):
    n = x.shape[0]
    e = edge_index.shape[1]
    cs = 128
    blk = NTILE * cs
    n_pad = ((n + blk - 1) // blk) * blk
    ept = ((e + NTILE * cs - 1) // (NTILE * cs)) * cs
    nchunk = ept // cs
    e_pad = ept * NTILE

    src = jnp.concatenate(
        [edge_index[0], jnp.zeros((e_pad - e,), edge_index.dtype)]
    ).astype(jnp.int32).reshape(NTILE, nchunk, cs)
    dst = jnp.concatenate(
        [edge_index[1], jnp.zeros((e_pad - e,), edge_index.dtype)]
    ).astype(jnp.int32).reshape(NTILE, nchunk, cs)

    xp = jnp.pad(x, ((0, n_pad - n), (0, 0)))
    d0 = x.shape[1] // 2
    h = jnp.stack([xp[:, :d0], xp[:, d0:]])
    for W, al, ar, b, nhalf in ((W1, al1, ar1, b1, 1), (W2, al2, ar2, b2, 2),
                                (W3, al3, ar3, b3, 4)):
        f = al.shape[1]
        fa = f // nhalf
        feat2d, el, er = _feat_el_er(h, W, al, ar, n_pad, nhalf)
        h = _sc_layer(feat2d, el, er, b.reshape(NSC * nhalf, fa), src, dst,
                      n_pad, e, nchunk, cs, nhalf)
    return _pool_heads(h, gw1, gb1, gw2, gb2, mu_w, mu_b, lv_w, lv_b, n)
